# baseline (speedup 1.0000x reference)
import functools

import jax
import jax.numpy as jnp
from jax import lax
from jax.experimental import pallas as pl
from jax.experimental.pallas import tpu as pltpu
from jax.experimental.pallas import tpu_sc as plsc

TOPK = 2
TM_ROUTER = 512
TM_EXPERT = 256
TM_COMBINE = 512
NUM_SC_CORES = 2
NUM_SC_SUBCORES = 16
NUM_SC_WORKERS = NUM_SC_CORES * NUM_SC_SUBCORES


def _router_body(x_ref, w1_ref, b1_ref, w2_ref, b2_ref,
                 eid_ref, wsel_ref, pos_ref, cnt_ref, xbf_ref, carry_ref):
    i = pl.program_id(0)

    @pl.when(i == 0)
    def _init():
        carry_ref[...] = jnp.zeros_like(carry_ref)

    x = x_ref[...]
    h = jnp.dot(x, w1_ref[...], preferred_element_type=jnp.float32) + b1_ref[...]
    h = h * jax.nn.sigmoid(h)
    rv = jnp.dot(h, w2_ref[...], preferred_element_type=jnp.float32) + b2_ref[...]
    tm, e = rv.shape

    ids = lax.broadcasted_iota(jnp.int32, (tm, e), 1)
    v1 = jnp.max(rv, axis=1, keepdims=True)
    idx1 = jnp.min(jnp.where(rv == v1, ids, e), axis=1, keepdims=True)
    m1 = ids == idx1
    rv2 = jnp.where(m1, -jnp.inf, rv)
    v2 = jnp.max(rv2, axis=1, keepdims=True)
    idx2 = jnp.min(jnp.where(rv2 == v2, ids, e), axis=1, keepdims=True)
    m2 = ids == idx2

    s = jnp.exp(v2 - v1)
    w1v = 1.0 / (1.0 + s)
    w2v = 1.0 - w1v

    maskf = (m1 | m2).astype(jnp.float32)
    r_ids = lax.broadcasted_iota(jnp.int32, (tm, tm), 0)
    c_ids = lax.broadcasted_iota(jnp.int32, (tm, tm), 1)
    tri = (c_ids < r_ids).astype(jnp.float32)
    pos_excl = jnp.dot(tri, maskf, preferred_element_type=jnp.float32)
    carry = carry_ref[...]
    pos = carry + pos_excl
    carry = carry + jnp.sum(maskf, axis=0, keepdims=True)
    carry_ref[...] = carry
    cnt_ref[...] = carry

    m1f = m1.astype(jnp.float32)
    m2f = m2.astype(jnp.float32)
    p1 = jnp.sum(pos * m1f, axis=1, keepdims=True)
    p2 = jnp.sum(pos * m2f, axis=1, keepdims=True)

    eid_ref[...] = jnp.concatenate([idx1, idx2], axis=1)
    wsel_ref[...] = jnp.concatenate([w1v, w2v], axis=1)
    pos_ref[...] = jnp.concatenate([p1, p2], axis=1).astype(jnp.int32)
    xbf_ref[...] = x.astype(jnp.bfloat16)


def _router(xf, W1, b1, W2, b2):
    T, D = xf.shape
    E = W2.shape[1]
    tm = TM_ROUTER
    return pl.pallas_call(
        _router_body,
        grid=(T // tm,),
        in_specs=[
            pl.BlockSpec((tm, D), lambda i: (i, 0)),
            pl.BlockSpec((D, D), lambda i: (0, 0)),
            pl.BlockSpec((1, D), lambda i: (0, 0)),
            pl.BlockSpec((D, E), lambda i: (0, 0)),
            pl.BlockSpec((1, E), lambda i: (0, 0)),
        ],
        out_specs=[
            pl.BlockSpec((tm, TOPK), lambda i: (i, 0)),
            pl.BlockSpec((tm, TOPK), lambda i: (i, 0)),
            pl.BlockSpec((tm, TOPK), lambda i: (i, 0)),
            pl.BlockSpec((1, E), lambda i: (0, 0)),
            pl.BlockSpec((tm, D), lambda i: (i, 0)),
        ],
        out_shape=[
            jax.ShapeDtypeStruct((T, TOPK), jnp.int32),
            jax.ShapeDtypeStruct((T, TOPK), jnp.float32),
            jax.ShapeDtypeStruct((T, TOPK), jnp.int32),
            jax.ShapeDtypeStruct((1, E), jnp.float32),
            jax.ShapeDtypeStruct((T, D), jnp.bfloat16),
        ],
        scratch_shapes=[pltpu.VMEM((1, E), jnp.float32)],
    )(xf, W1, b1.reshape(1, D), W2, b2.reshape(1, E))


def _sc_gather_rows_i32(table, idx):
    R, Cw = table.shape
    (N,) = idx.shape
    n_per_w = N // NUM_SC_WORKERS
    chunk = 8
    for c in range(n_per_w, 7, -8):
        if n_per_w % c == 0 and c % 8 == 0 and c * Cw * 4 <= 256 * 1024:
            chunk = c
            break
    n_steps = n_per_w // chunk
    mesh = plsc.VectorSubcoreMesh(core_axis_name="c", subcore_axis_name="s")

    @functools.partial(
        pl.kernel,
        mesh=mesh,
        out_type=jax.ShapeDtypeStruct((N, Cw), jnp.int32),
        scratch_types=[
            pltpu.VMEM((chunk,), jnp.int32),
            pltpu.VMEM((chunk, Cw), jnp.int32),
            pltpu.SemaphoreType.DMA,
        ],
    )
    def k(table_hbm, idx_hbm, out_hbm, idx_v, rows_v, sem):
        wid = lax.axis_index("s") * NUM_SC_CORES + lax.axis_index("c")
        base = wid * n_per_w

        def body(c, carry):
            off = base + c * chunk
            pltpu.sync_copy(idx_hbm.at[pl.ds(off, chunk)], idx_v)
            pltpu.async_copy(table_hbm.at[idx_v], rows_v, sem).wait()
            pltpu.sync_copy(rows_v, out_hbm.at[pl.ds(off, chunk)])
            return carry

        lax.fori_loop(0, n_steps, body, 0)

    return k(table, idx)


def _expert_body(eid_pref, xg_ref, we_ref, be_ref, out_ref):
    acc = jnp.dot(xg_ref[...], we_ref[0], preferred_element_type=jnp.float32)
    out_ref[...] = acc + be_ref[0]


def _grouped_expert_matmul(xg, We_bf, be, tile_eid):
    L, D = xg.shape
    E = We_bf.shape[0]
    tm = TM_EXPERT
    nt = L // tm
    grid_spec = pltpu.PrefetchScalarGridSpec(
        num_scalar_prefetch=1,
        grid=(nt,),
        in_specs=[
            pl.BlockSpec((tm, D), lambda i, eid: (i, 0)),
            pl.BlockSpec((1, D, D), lambda i, eid: (eid[i], 0, 0)),
            pl.BlockSpec((1, 1, D), lambda i, eid: (eid[i], 0, 0)),
        ],
        out_specs=pl.BlockSpec((tm, D), lambda i, eid: (i, 0)),
    )
    return pl.pallas_call(
        _expert_body,
        grid_spec=grid_spec,
        out_shape=jax.ShapeDtypeStruct((L, D), jnp.float32),
    )(tile_eid, xg, We_bf, be.reshape(E, 1, D))


def _combine_body(x_ref, g0_ref, g1_ref, w0_ref, w1_ref, y_ref):
    y_ref[...] = (x_ref[...]
                  + w0_ref[...] * g0_ref[...]
                  + w1_ref[...] * g1_ref[...])


def _combine(xf, g, wsel):
    T, D = xf.shape
    tm = TM_COMBINE
    nb = T // tm
    w0 = wsel[:, 0:1]
    w1 = wsel[:, 1:2]
    return pl.pallas_call(
        _combine_body,
        grid=(nb,),
        in_specs=[
            pl.BlockSpec((tm, D), lambda i: (i, 0)),
            pl.BlockSpec((tm, D), lambda i: (i, 0)),
            pl.BlockSpec((tm, D), lambda i: (i + nb, 0)),
            pl.BlockSpec((tm, 1), lambda i: (i, 0)),
            pl.BlockSpec((tm, 1), lambda i: (i, 0)),
        ],
        out_specs=pl.BlockSpec((tm, D), lambda i: (i, 0)),
        out_shape=jax.ShapeDtypeStruct((T, D), jnp.float32),
    )(xf, g, g, w0, w1)


def kernel(x, W1, b1, W2, b2, We, be):
    B, S, D = x.shape
    E = W2.shape[1]
    T = B * S
    xf = x.reshape(T, D)

    eid, wsel, pos, cnt, xbf = _router(xf, W1, b1, W2, b2)

    tm = TM_EXPERT
    L = TOPK * T + E * tm
    nt = L // tm
    counts = cnt[0].astype(jnp.int32)
    padded = ((counts + tm - 1) // tm) * tm
    ends = jnp.cumsum(padded)
    offsets = ends - padded
    dest = offsets[eid] + pos
    end_tiles = ends // tm
    tile_ids = jnp.arange(nt, dtype=jnp.int32)
    tile_eid = jnp.minimum(
        jnp.sum((tile_ids[:, None] >= end_tiles[None, :]).astype(jnp.int32), axis=1),
        E - 1).astype(jnp.int32)
    src_token = jnp.zeros((L,), jnp.int32).at[dest.reshape(-1)].set(
        jnp.repeat(jnp.arange(T, dtype=jnp.int32), TOPK))

    xbf_i32 = lax.bitcast_convert_type(xbf.reshape(T, D // 2, 2), jnp.int32)
    xg_i32 = _sc_gather_rows_i32(xbf_i32, src_token)
    xg = lax.bitcast_convert_type(xg_i32, jnp.bfloat16).reshape(L, D)

    eout = _grouped_expert_matmul(xg, We.astype(jnp.bfloat16), be, tile_eid)

    eout_i32 = lax.bitcast_convert_type(eout, jnp.int32)
    dflat = dest.T.reshape(TOPK * T)
    g_i32 = _sc_gather_rows_i32(eout_i32, dflat)
    g = lax.bitcast_convert_type(g_i32, jnp.float32)

    y = _combine(xf, g, wsel)
    return y.reshape(B, S, D)


# speedup vs baseline: 2.1026x; 2.1026x over previous
import functools

import jax
import jax.numpy as jnp
from jax import lax
from jax.experimental import pallas as pl
from jax.experimental.pallas import tpu as pltpu
from jax.experimental.pallas import tpu_sc as plsc

TOPK = 2
TM_ROUTER = 512
TM_EXPERT = 256
TM_COMBINE = 512
NUM_SC_CORES = 2
NUM_SC_SUBCORES = 16
NUM_SC_WORKERS = NUM_SC_CORES * NUM_SC_SUBCORES


def _router_body(x_ref, w1_ref, b1_ref, w2_ref, b2_ref,
                 eid_ref, wsel_ref, pos_ref, cnt_ref, carry_ref):
    i = pl.program_id(0)

    @pl.when(i == 0)
    def _init():
        carry_ref[...] = jnp.zeros_like(carry_ref)

    x = x_ref[...]
    h = jnp.dot(x, w1_ref[...], preferred_element_type=jnp.float32) + b1_ref[...]
    h = h * jax.nn.sigmoid(h)
    rv = jnp.dot(h, w2_ref[...], preferred_element_type=jnp.float32) + b2_ref[...]
    tm, e = rv.shape

    ids = lax.broadcasted_iota(jnp.int32, (tm, e), 1)
    v1 = jnp.max(rv, axis=1, keepdims=True)
    idx1 = jnp.min(jnp.where(rv == v1, ids, e), axis=1, keepdims=True)
    m1 = ids == idx1
    rv2 = jnp.where(m1, -jnp.inf, rv)
    v2 = jnp.max(rv2, axis=1, keepdims=True)
    idx2 = jnp.min(jnp.where(rv2 == v2, ids, e), axis=1, keepdims=True)
    m2 = ids == idx2

    s = jnp.exp(v2 - v1)
    w1v = 1.0 / (1.0 + s)
    w2v = 1.0 - w1v

    maskf = (m1 | m2).astype(jnp.float32)
    r_ids = lax.broadcasted_iota(jnp.int32, (tm, tm), 0)
    c_ids = lax.broadcasted_iota(jnp.int32, (tm, tm), 1)
    tri = (c_ids < r_ids).astype(jnp.float32)
    pos_excl = jnp.dot(tri, maskf, preferred_element_type=jnp.float32)
    carry = carry_ref[...]
    pos = carry + pos_excl
    carry = carry + jnp.sum(maskf, axis=0, keepdims=True)
    carry_ref[...] = carry
    cnt_ref[...] = carry

    m1f = m1.astype(jnp.float32)
    m2f = m2.astype(jnp.float32)
    p1 = jnp.sum(pos * m1f, axis=1, keepdims=True)
    p2 = jnp.sum(pos * m2f, axis=1, keepdims=True)

    eid_ref[...] = jnp.concatenate([idx1, idx2], axis=1)
    wsel_ref[...] = jnp.concatenate([w1v, w2v], axis=1)
    pos_ref[...] = jnp.concatenate([p1, p2], axis=1).astype(jnp.int32)


def _router(xf, W1, b1, W2, b2):
    T, D = xf.shape
    E = W2.shape[1]
    tm = TM_ROUTER
    return pl.pallas_call(
        _router_body,
        grid=(T // tm,),
        in_specs=[
            pl.BlockSpec((tm, D), lambda i: (i, 0)),
            pl.BlockSpec((D, D), lambda i: (0, 0)),
            pl.BlockSpec((1, D), lambda i: (0, 0)),
            pl.BlockSpec((D, E), lambda i: (0, 0)),
            pl.BlockSpec((1, E), lambda i: (0, 0)),
        ],
        out_specs=[
            pl.BlockSpec((tm, TOPK), lambda i: (i, 0)),
            pl.BlockSpec((tm, TOPK), lambda i: (i, 0)),
            pl.BlockSpec((tm, TOPK), lambda i: (i, 0)),
            pl.BlockSpec((1, E), lambda i: (0, 0)),
        ],
        out_shape=[
            jax.ShapeDtypeStruct((T, TOPK), jnp.int32),
            jax.ShapeDtypeStruct((T, TOPK), jnp.float32),
            jax.ShapeDtypeStruct((T, TOPK), jnp.int32),
            jax.ShapeDtypeStruct((1, E), jnp.float32),
        ],
        scratch_shapes=[pltpu.VMEM((1, E), jnp.float32)],
    )(xf, W1, b1.reshape(1, D), W2, b2.reshape(1, E))


def _sc_gather_rows(table, idx):
    R, Cw = table.shape
    (N,) = idx.shape
    itemsize = jnp.dtype(table.dtype).itemsize
    n_per_w = N // NUM_SC_WORKERS
    chunk = 8
    for c in range(n_per_w, 7, -8):
        if n_per_w % c == 0 and c % 8 == 0 and c * Cw * itemsize <= 192 * 1024:
            chunk = c
            break
    n_steps = n_per_w // chunk
    mesh = plsc.VectorSubcoreMesh(core_axis_name="c", subcore_axis_name="s")

    @functools.partial(
        pl.kernel,
        mesh=mesh,
        out_type=jax.ShapeDtypeStruct((N, Cw), table.dtype),
        scratch_types=[
            pltpu.VMEM((2, chunk), jnp.int32),
            pltpu.VMEM((2, chunk, Cw), table.dtype),
            pltpu.SemaphoreType.DMA,
            pltpu.SemaphoreType.DMA,
            pltpu.SemaphoreType.DMA,
            pltpu.SemaphoreType.DMA,
        ],
    )
    def k(table_hbm, idx_hbm, out_hbm, idx_v, rows_v, sg0, sg1, sw0, sw1):
        wid = lax.axis_index("s") * NUM_SC_CORES + lax.axis_index("c")
        base = wid * n_per_w
        sg = [sg0, sg1]
        sw = [sw0, sw1]

        pltpu.sync_copy(idx_hbm.at[pl.ds(base, chunk)], idx_v.at[0])
        gathers = [pltpu.async_copy(table_hbm.at[idx_v.at[0]], rows_v.at[0], sg[0]),
                   None]
        writes = [None, None]
        for c in range(n_steps):
            cur = c & 1
            nxt = cur ^ 1
            if c + 1 < n_steps:
                off = base + (c + 1) * chunk
                pltpu.sync_copy(idx_hbm.at[pl.ds(off, chunk)], idx_v.at[nxt])
                if writes[nxt] is not None:
                    writes[nxt].wait()
                    writes[nxt] = None
                gathers[nxt] = pltpu.async_copy(
                    table_hbm.at[idx_v.at[nxt]], rows_v.at[nxt], sg[nxt])
            gathers[cur].wait()
            writes[cur] = pltpu.async_copy(
                rows_v.at[cur], out_hbm.at[pl.ds(base + c * chunk, chunk)], sw[cur])
        for w in writes:
            if w is not None:
                w.wait()

    return k(table, idx)


def _expert_body(eid_pref, xg_ref, we_ref, be_ref, out_ref):
    acc = jnp.dot(xg_ref[...].astype(jnp.bfloat16), we_ref[0],
                  preferred_element_type=jnp.float32)
    out_ref[...] = acc + be_ref[0]


def _grouped_expert_matmul(xg, We_bf, be, tile_eid):
    L, D = xg.shape
    E = We_bf.shape[0]
    tm = TM_EXPERT
    nt = L // tm
    grid_spec = pltpu.PrefetchScalarGridSpec(
        num_scalar_prefetch=1,
        grid=(nt,),
        in_specs=[
            pl.BlockSpec((tm, D), lambda i, eid: (i, 0)),
            pl.BlockSpec((1, D, D), lambda i, eid: (eid[i], 0, 0)),
            pl.BlockSpec((1, 1, D), lambda i, eid: (eid[i], 0, 0)),
        ],
        out_specs=pl.BlockSpec((tm, D), lambda i, eid: (i, 0)),
    )
    return pl.pallas_call(
        _expert_body,
        grid_spec=grid_spec,
        out_shape=jax.ShapeDtypeStruct((L, D), jnp.float32),
    )(tile_eid, xg, We_bf, be.reshape(E, 1, D))


def _combine_body(x_ref, g0_ref, g1_ref, w0_ref, w1_ref, y_ref):
    y_ref[...] = (x_ref[...]
                  + w0_ref[...] * g0_ref[...]
                  + w1_ref[...] * g1_ref[...])


def _combine(xf, g, wsel):
    T, D = xf.shape
    tm = TM_COMBINE
    nb = T // tm
    w0 = wsel[:, 0:1]
    w1 = wsel[:, 1:2]
    return pl.pallas_call(
        _combine_body,
        grid=(nb,),
        in_specs=[
            pl.BlockSpec((tm, D), lambda i: (i, 0)),
            pl.BlockSpec((tm, D), lambda i: (i, 0)),
            pl.BlockSpec((tm, D), lambda i: (i + nb, 0)),
            pl.BlockSpec((tm, 1), lambda i: (i, 0)),
            pl.BlockSpec((tm, 1), lambda i: (i, 0)),
        ],
        out_specs=pl.BlockSpec((tm, D), lambda i: (i, 0)),
        out_shape=jax.ShapeDtypeStruct((T, D), jnp.float32),
    )(xf, g, g, w0, w1)


def kernel(x, W1, b1, W2, b2, We, be):
    B, S, D = x.shape
    E = W2.shape[1]
    T = B * S
    xf = x.reshape(T, D)

    eid, wsel, pos, cnt = _router(xf, W1, b1, W2, b2)

    tm = TM_EXPERT
    L = TOPK * T + E * tm
    nt = L // tm
    counts = cnt[0].astype(jnp.int32)
    padded = ((counts + tm - 1) // tm) * tm
    ends = jnp.cumsum(padded)
    offsets = ends - padded
    dest = offsets[eid] + pos
    end_tiles = ends // tm
    tile_ids = jnp.arange(nt, dtype=jnp.int32)
    tile_eid = jnp.minimum(
        jnp.sum((tile_ids[:, None] >= end_tiles[None, :]).astype(jnp.int32), axis=1),
        E - 1).astype(jnp.int32)
    src_token = jnp.zeros((L,), jnp.int32).at[dest.reshape(-1)].set(
        jnp.repeat(jnp.arange(T, dtype=jnp.int32), TOPK))

    xg = _sc_gather_rows(xf, src_token)

    eout = _grouped_expert_matmul(xg, We.astype(jnp.bfloat16), be, tile_eid)

    dflat = dest.T.reshape(TOPK * T)
    g = _sc_gather_rows(eout, dflat)

    y = _combine(xf, g, wsel)
    return y.reshape(B, S, D)


# speedup vs baseline: 2.3182x; 1.1026x over previous
import functools

import jax
import jax.numpy as jnp
from jax import lax
from jax.experimental import pallas as pl
from jax.experimental.pallas import tpu as pltpu
from jax.experimental.pallas import tpu_sc as plsc

TOPK = 2
TM_ROUTER = 512
TM_EXPERT = 256
TM_COMBINE = 512
NUM_SC_CORES = 2
NUM_SC_SUBCORES = 16
NUM_SC_WORKERS = NUM_SC_CORES * NUM_SC_SUBCORES


def _router_body(x_ref, w1_ref, b1_ref, w2_ref, b2_ref,
                 eid_ref, wsel_ref, pos_ref, cnt_ref, carry_ref):
    i = pl.program_id(0)

    @pl.when(i == 0)
    def _init():
        carry_ref[...] = jnp.zeros_like(carry_ref)

    x = x_ref[...]
    h = jnp.dot(x, w1_ref[...], preferred_element_type=jnp.float32) + b1_ref[...]
    h = h * jax.nn.sigmoid(h)
    rv = jnp.dot(h, w2_ref[...], preferred_element_type=jnp.float32) + b2_ref[...]
    tm, e = rv.shape

    ids = lax.broadcasted_iota(jnp.int32, (tm, e), 1)
    v1 = jnp.max(rv, axis=1, keepdims=True)
    idx1 = jnp.min(jnp.where(rv == v1, ids, e), axis=1, keepdims=True)
    m1 = ids == idx1
    rv2 = jnp.where(m1, -jnp.inf, rv)
    v2 = jnp.max(rv2, axis=1, keepdims=True)
    idx2 = jnp.min(jnp.where(rv2 == v2, ids, e), axis=1, keepdims=True)
    m2 = ids == idx2

    s = jnp.exp(v2 - v1)
    w1v = 1.0 / (1.0 + s)
    w2v = 1.0 - w1v

    maskf = (m1 | m2).astype(jnp.float32)
    r_ids = lax.broadcasted_iota(jnp.int32, (tm, tm), 0)
    c_ids = lax.broadcasted_iota(jnp.int32, (tm, tm), 1)
    tri = (c_ids < r_ids).astype(jnp.float32)
    pos_excl = jnp.dot(tri, maskf, preferred_element_type=jnp.float32)
    carry = carry_ref[...]
    pos = carry + pos_excl
    carry = carry + jnp.sum(maskf, axis=0, keepdims=True)
    carry_ref[...] = carry
    cnt_ref[...] = carry

    m1f = m1.astype(jnp.float32)
    m2f = m2.astype(jnp.float32)
    p1 = jnp.sum(pos * m1f, axis=1, keepdims=True)
    p2 = jnp.sum(pos * m2f, axis=1, keepdims=True)

    eid_ref[...] = jnp.concatenate([idx1, idx2], axis=1)
    wsel_ref[...] = jnp.concatenate([w1v, w2v], axis=1)
    pos_ref[...] = jnp.concatenate([p1, p2], axis=1).astype(jnp.int32)


def _router(xf, W1, b1, W2, b2):
    T, D = xf.shape
    E = W2.shape[1]
    tm = TM_ROUTER
    return pl.pallas_call(
        _router_body,
        grid=(T // tm,),
        in_specs=[
            pl.BlockSpec((tm, D), lambda i: (i, 0)),
            pl.BlockSpec((D, D), lambda i: (0, 0)),
            pl.BlockSpec((1, D), lambda i: (0, 0)),
            pl.BlockSpec((D, E), lambda i: (0, 0)),
            pl.BlockSpec((1, E), lambda i: (0, 0)),
        ],
        out_specs=[
            pl.BlockSpec((tm, TOPK), lambda i: (i, 0)),
            pl.BlockSpec((tm, TOPK), lambda i: (i, 0)),
            pl.BlockSpec((tm, TOPK), lambda i: (i, 0)),
            pl.BlockSpec((1, E), lambda i: (0, 0)),
        ],
        out_shape=[
            jax.ShapeDtypeStruct((T, TOPK), jnp.int32),
            jax.ShapeDtypeStruct((T, TOPK), jnp.float32),
            jax.ShapeDtypeStruct((T, TOPK), jnp.int32),
            jax.ShapeDtypeStruct((1, E), jnp.float32),
        ],
        scratch_shapes=[pltpu.VMEM((1, E), jnp.float32)],
    )(xf, W1, b1.reshape(1, D), W2, b2.reshape(1, E))


def _sc_gather_rows(table, idx):
    row_shape = table.shape[1:]
    row_elems = 1
    for s in row_shape:
        row_elems *= s
    (N,) = idx.shape
    itemsize = jnp.dtype(table.dtype).itemsize
    n_per_w = N // NUM_SC_WORKERS
    chunk = 8
    for c in range(n_per_w, 7, -8):
        if n_per_w % c == 0 and c % 8 == 0 and c * row_elems * itemsize <= 192 * 1024:
            chunk = c
            break
    n_steps = n_per_w // chunk
    mesh = plsc.VectorSubcoreMesh(core_axis_name="c", subcore_axis_name="s")

    @functools.partial(
        pl.kernel,
        mesh=mesh,
        out_type=jax.ShapeDtypeStruct((N,) + row_shape, table.dtype),
        scratch_types=[
            pltpu.VMEM((2, chunk), jnp.int32),
            pltpu.VMEM((2, chunk) + row_shape, table.dtype),
            pltpu.SemaphoreType.DMA,
            pltpu.SemaphoreType.DMA,
            pltpu.SemaphoreType.DMA,
            pltpu.SemaphoreType.DMA,
        ],
    )
    def k(table_hbm, idx_hbm, out_hbm, idx_v, rows_v, sg0, sg1, sw0, sw1):
        wid = lax.axis_index("s") * NUM_SC_CORES + lax.axis_index("c")
        base = wid * n_per_w
        sg = [sg0, sg1]
        sw = [sw0, sw1]

        pltpu.sync_copy(idx_hbm.at[pl.ds(base, chunk)], idx_v.at[0])
        gathers = [pltpu.async_copy(table_hbm.at[idx_v.at[0]], rows_v.at[0], sg[0]),
                   None]
        writes = [None, None]
        for c in range(n_steps):
            cur = c & 1
            nxt = cur ^ 1
            if c + 1 < n_steps:
                off = base + (c + 1) * chunk
                pltpu.sync_copy(idx_hbm.at[pl.ds(off, chunk)], idx_v.at[nxt])
                if writes[nxt] is not None:
                    writes[nxt].wait()
                    writes[nxt] = None
                gathers[nxt] = pltpu.async_copy(
                    table_hbm.at[idx_v.at[nxt]], rows_v.at[nxt], sg[nxt])
            gathers[cur].wait()
            writes[cur] = pltpu.async_copy(
                rows_v.at[cur], out_hbm.at[pl.ds(base + c * chunk, chunk)], sw[cur])
        for w in writes:
            if w is not None:
                w.wait()

    return k(table, idx)


def _expert_body(eid_pref, xg_ref, we_ref, be_ref, out_ref):
    nk = xg_ref.shape[1]
    lhs = jnp.concatenate([xg_ref[:, k, :] for k in range(nk)], axis=1)
    acc = jnp.dot(lhs.astype(jnp.bfloat16), we_ref[0],
                  preferred_element_type=jnp.float32)
    out_ref[...] = acc + be_ref[0]


def _expert_body_alias(eid_pref, xg_ref, we_ref, be_ref, prev_ref, out_ref):
    _expert_body(eid_pref, xg_ref, we_ref, be_ref, out_ref)


def _grouped_expert_matmul(xg_half, We_bf, be, tile_eid, L, tile0, prev=None):
    n_half, nk, kb = xg_half.shape
    D = nk * kb
    E = We_bf.shape[0]
    tm = TM_EXPERT
    nt = n_half // tm
    grid_spec = pltpu.PrefetchScalarGridSpec(
        num_scalar_prefetch=1,
        grid=(nt,),
        in_specs=[
            pl.BlockSpec((tm, nk, kb), lambda i, eid: (i, 0, 0)),
            pl.BlockSpec((1, D, D), lambda i, eid: (eid[i + tile0], 0, 0)),
            pl.BlockSpec((1, 1, D), lambda i, eid: (eid[i + tile0], 0, 0)),
        ] + ([pl.BlockSpec(memory_space=pl.ANY)] if prev is not None else []),
        out_specs=pl.BlockSpec((tm, D), lambda i, eid: (i + tile0, 0)),
    )
    if prev is None:
        return pl.pallas_call(
            _expert_body,
            grid_spec=grid_spec,
            out_shape=jax.ShapeDtypeStruct((L, D), jnp.float32),
        )(tile_eid, xg_half, We_bf, be.reshape(E, 1, D))
    return pl.pallas_call(
        _expert_body_alias,
        grid_spec=grid_spec,
        out_shape=jax.ShapeDtypeStruct((L, D), jnp.float32),
        input_output_aliases={4: 0},
    )(tile_eid, xg_half, We_bf, be.reshape(E, 1, D), prev)


def _combine_body(x_ref, g0_ref, g1_ref, w0_ref, w1_ref, y_ref):
    y_ref[...] = (x_ref[...]
                  + w0_ref[...] * g0_ref[...]
                  + w1_ref[...] * g1_ref[...])


def _combine(xf, g, wsel):
    T, D = xf.shape
    tm = TM_COMBINE
    nb = T // tm
    w0 = wsel[:, 0:1]
    w1 = wsel[:, 1:2]
    return pl.pallas_call(
        _combine_body,
        grid=(nb,),
        in_specs=[
            pl.BlockSpec((tm, D), lambda i: (i, 0)),
            pl.BlockSpec((tm, D), lambda i: (i, 0)),
            pl.BlockSpec((tm, D), lambda i: (i + nb, 0)),
            pl.BlockSpec((tm, 1), lambda i: (i, 0)),
            pl.BlockSpec((tm, 1), lambda i: (i, 0)),
        ],
        out_specs=pl.BlockSpec((tm, D), lambda i: (i, 0)),
        out_shape=jax.ShapeDtypeStruct((T, D), jnp.float32),
    )(xf, g, g, w0, w1)


def kernel(x, W1, b1, W2, b2, We, be):
    B, S, D = x.shape
    E = W2.shape[1]
    T = B * S
    xf = x.reshape(T, D)

    eid, wsel, pos, cnt = _router(xf, W1, b1, W2, b2)

    tm = TM_EXPERT
    L = TOPK * T + E * tm
    nt = L // tm
    counts = cnt[0].astype(jnp.int32)
    padded = ((counts + tm - 1) // tm) * tm
    ends = jnp.cumsum(padded)
    offsets = ends - padded
    dest = offsets[eid] + pos
    end_tiles = ends // tm
    tile_ids = jnp.arange(nt, dtype=jnp.int32)
    tile_eid = jnp.minimum(
        jnp.sum((tile_ids[:, None] >= end_tiles[None, :]).astype(jnp.int32), axis=1),
        E - 1).astype(jnp.int32)
    src_token = jnp.zeros((L,), jnp.int32).at[dest.reshape(-1)].set(
        jnp.repeat(jnp.arange(T, dtype=jnp.int32), TOPK))

    x3 = jnp.reshape(xf, (T, D // 128, 128))
    Lh = L // 2
    nth = Lh // tm
    We_bf = We.astype(jnp.bfloat16)
    xgA = _sc_gather_rows(x3, src_token[:Lh])
    eoutA = _grouped_expert_matmul(xgA, We_bf, be, tile_eid, L, 0)
    xgB = _sc_gather_rows(x3, src_token[Lh:])
    eout = _grouped_expert_matmul(xgB, We_bf, be, tile_eid, L, nth, prev=eoutA)

    dflat = dest.T.reshape(TOPK * T)
    g = _sc_gather_rows(eout, dflat)

    y = _combine(xf, g, wsel)
    return y.reshape(B, S, D)
